# Initial kernel scaffold; baseline (speedup 1.0000x reference)
import jax
import jax.numpy as jnp
from jax.experimental import pallas as pl


def kernel(X, C, W_node, b_node, W_edge, b_edge):
    raise NotImplementedError("write your pallas kernel here")


# speedup vs baseline: 19.7581x; 19.7581x over previous
import jax
import jax.numpy as jnp
from jax.experimental import pallas as pl

R = 512
TI = 64
NUM_RBF = 32
MU_STEP = 20.0 / (NUM_RBF - 1)
INV_SIGMA = NUM_RBF / 20.0


def _fused_kernel(x_ref, c_ref, wn_ref, bn_ref, w_ref,
                  nh_ref, mcol_ref, mrow_ref, xc_ref, xct_ref,
                  eh_ref, mij_ref, idx_ref):
    i = pl.program_id(0)

    @pl.when(i == 0)
    def _node_path():
        x = x_ref[...]
        dX = x[:, 1:, :] - x[:, :-1, :]
        l = jnp.sqrt(jnp.sum(dX * dX, axis=-1))
        log_len = jnp.log(l + 1e-6)
        u = dX / (l + 1e-6)[..., None]
        feat = jnp.concatenate(
            [log_len, u[:, 0, :], u[:, 1, :], u[:, 2, :]], axis=-1)
        m = (c_ref[...] > 0).astype(jnp.float32)
        nh = jnp.dot(feat, wn_ref[...], preferred_element_type=jnp.float32)
        nh_ref[...] = (nh + bn_ref[...]) * m
        mcol_ref[...] = m
        mrow_ref[...] = m.reshape(1, R)
        xc = jnp.mean(x, axis=1)
        xc_ref[...] = xc
        xct_ref[...] = xc.T

    base = i * TI
    xi = xc_ref[pl.ds(base, TI), :]
    xjt = xct_ref[...]
    dx = xjt[0:1, :] - xi[:, 0:1]
    dy = xjt[1:2, :] - xi[:, 1:2]
    dz = xjt[2:3, :] - xi[:, 2:3]
    m = mcol_ref[pl.ds(base, TI), :] * mrow_ref[...]
    mij_ref[...] = m
    idx_ref[...] = jax.lax.broadcasted_iota(jnp.int32, (TI, R), 1)
    d2 = dx * dx + dy * dy + dz * dz
    d = jnp.sqrt(d2)
    rinv = 1.0 / (d + 1e-6)
    uxm = dx * rinv * m
    uym = dy * rinv * m
    uzm = dz * rinv * m
    neg_big = (m - 1.0) * 1e30
    mu = jax.lax.broadcasted_iota(
        jnp.int32, (1, NUM_RBF, 1), 1).astype(jnp.float32) * MU_STEP
    t = (d[:, None, :] - mu) * INV_SIGMA
    rbf = jnp.exp(neg_big[:, None, :] - t * t)
    feat = jnp.concatenate(
        [rbf, uxm[:, None, :], uym[:, None, :], uzm[:, None, :],
         m[:, None, :]], axis=1)
    out = jax.lax.dot_general(
        feat, w_ref[...], (((1,), (0,)), ((), ())),
        preferred_element_type=jnp.float32)
    eh_ref[...] = out


def kernel(X, C, W_node, b_node, W_edge, b_edge):
    B = X.shape[0]
    x = X.reshape(R, 4, 3)
    c_col = C.reshape(R, 1)
    bn = b_node.reshape(1, -1)
    dim_nodes = W_node.shape[1]
    dim_edges = W_edge.shape[1]

    w36 = jnp.concatenate([W_edge, b_edge[None, :]], axis=0)

    nblk = R // TI
    const = lambda i: (0, 0)
    outs = pl.pallas_call(
        _fused_kernel,
        grid=(nblk,),
        in_specs=[
            pl.BlockSpec((R, 4, 3), lambda i: (0, 0, 0)),
            pl.BlockSpec((R, 1), const),
            pl.BlockSpec((12, dim_nodes), const),
            pl.BlockSpec((1, dim_nodes), const),
            pl.BlockSpec((NUM_RBF + 4, dim_edges), const),
        ],
        out_specs=(
            pl.BlockSpec((R, dim_nodes), const),
            pl.BlockSpec((R, 1), const),
            pl.BlockSpec((1, R), const),
            pl.BlockSpec((R, 3), const),
            pl.BlockSpec((3, R), const),
            pl.BlockSpec((TI, R, dim_edges), lambda i: (i, 0, 0)),
            pl.BlockSpec((TI, R), lambda i: (i, 0)),
            pl.BlockSpec((TI, R), lambda i: (i, 0)),
        ),
        out_shape=(
            jax.ShapeDtypeStruct((R, dim_nodes), jnp.float32),
            jax.ShapeDtypeStruct((R, 1), jnp.float32),
            jax.ShapeDtypeStruct((1, R), jnp.float32),
            jax.ShapeDtypeStruct((R, 3), jnp.float32),
            jax.ShapeDtypeStruct((3, R), jnp.float32),
            jax.ShapeDtypeStruct((R, R, dim_edges), jnp.float32),
            jax.ShapeDtypeStruct((R, R), jnp.float32),
            jax.ShapeDtypeStruct((R, R), jnp.int32),
        ),
    )(x, c_col, W_node, bn, w36)
    node_h, _mcol, mrow, _xc, _xct, edge_h, mask_ij, edge_idx = outs

    return (node_h.reshape(B, R, dim_nodes),
            edge_h.reshape(B, R, R, dim_edges),
            edge_idx.reshape(B, R, R),
            mrow.reshape(B, R),
            mask_ij.reshape(B, R, R))
